# baseline (speedup 1.0000x reference)
import functools

import jax
import jax.numpy as jnp
from jax import lax
from jax.experimental import pallas as pl
from jax.experimental.pallas import tpu as pltpu
from jax.experimental.pallas import tpu_sc as plsc

_EMBED = 32
_IDX_BLK = 128


def _build_sc_call(batch):
    info = plsc.get_sparse_core_info()
    nc, ns, lanes = info.num_cores, info.num_subcores, info.num_lanes
    nw = nc * ns
    b_per_w = batch // nw
    n_idx_blocks = b_per_w // _IDX_BLK
    mesh = plsc.VectorSubcoreMesh(core_axis_name="c", subcore_axis_name="s")

    @functools.partial(
        pl.kernel,
        mesh=mesh,
        out_type=jax.ShapeDtypeStruct((batch,), jnp.float32),
        scratch_types=[
            pltpu.VMEM((n_idx_blocks, _IDX_BLK), jnp.int32),
            pltpu.VMEM((n_idx_blocks, _IDX_BLK), jnp.int32),
            pltpu.VMEM((b_per_w, _EMBED), jnp.float32),
            pltpu.VMEM((b_per_w, _EMBED), jnp.float32),
            pltpu.VMEM((b_per_w,), jnp.float32),
            pltpu.SemaphoreType.DMA,
        ],
        compiler_params=pltpu.CompilerParams(
            needs_layout_passes=False, use_tc_tiling_on_sc=False),
    )
    def sc_call(uidx_hbm, midx_hbm, uemb_hbm, memb_hbm, out_hbm,
                uidx_v, midx_v, urows_v, mrows_v, out_v, sem):
        wid = lax.axis_index("s") * nc + lax.axis_index("c")
        base = wid * b_per_w
        pltpu.sync_copy(uidx_hbm.at[wid], uidx_v)
        pltpu.sync_copy(midx_hbm.at[wid], midx_v)
        copies = []
        for j in range(n_idx_blocks):
            copies.append(pltpu.async_copy(
                uemb_hbm.at[uidx_v.at[j]],
                urows_v.at[pl.ds(j * _IDX_BLK, _IDX_BLK)], sem))
            copies.append(pltpu.async_copy(
                memb_hbm.at[midx_v.at[j]],
                mrows_v.at[pl.ds(j * _IDX_BLK, _IDX_BLK)], sem))
        for c in copies:
            c.wait()

        lane_iota = lax.iota(jnp.int32, lanes)

        def blk_body(blk, carry):
            row = blk * lanes + lane_iota
            acc = jnp.zeros((lanes,), jnp.float32)
            for d in range(_EMBED):
                col = jnp.full((lanes,), d, jnp.int32)
                uu = plsc.load_gather(urows_v, [row, col])
                mm = plsc.load_gather(mrows_v, [row, col])
                acc = acc + uu * mm
            out_v[pl.ds(blk * lanes, lanes)] = acc
            return carry

        lax.fori_loop(0, b_per_w // lanes, blk_body, 0)
        pltpu.sync_copy(out_v, out_hbm.at[pl.ds(base, b_per_w)])

    return sc_call, nw, n_idx_blocks


def kernel(user_movie_pair, user_embeddings, movie_embeddings):
    batch = user_movie_pair.shape[0]
    sc_call, nw, n_idx_blocks = _build_sc_call(batch)
    pair = user_movie_pair.astype(jnp.int32)
    uidx = pair[:, 0].reshape(nw, n_idx_blocks, _IDX_BLK)
    midx = pair[:, 1].reshape(nw, n_idx_blocks, _IDX_BLK)
    out = sc_call(uidx, midx, user_embeddings, movie_embeddings)
    return out.reshape(batch, 1)
